# baseline (speedup 1.0000x reference)
import functools

import jax
import jax.numpy as jnp
from jax import lax
from jax.experimental import pallas as pl
from jax.experimental.pallas import tpu as pltpu
from jax.experimental.pallas import tpu_sc as plsc

_H = 32
_NC = 2
_NS = 16
_CH = 16384
_NBUF = 6
_A = 3


def _sc_body(scal_hbm, hist_hbm, act_hbm, out_hbm, scal_v, *rest):
    bufs = list(rest[:_NBUF])
    in_sem, out_sem = rest[_NBUF], rest[_NBUF + 1]
    B, H, N = 16, _H, 65536
    ncr = N // _CH
    K = B * ncr

    wid = lax.axis_index("s") * _NC + lax.axis_index("c")
    pltpu.sync_copy(scal_hbm, scal_v)
    sv = scal_v[...]
    shift = sv[0]
    pos = sv[1]

    j = wid
    src = lax.rem(j + shift, _H)
    is_act = j == pos

    def start_in(k, slot):
        b = k // ncr
        base = (k % ncr) * _CH

        @pl.when(jnp.logical_not(is_act))
        def _():
            pltpu.make_async_copy(
                hist_hbm.at[b, src, pl.ds(base, _CH)], bufs[slot], in_sem.at[slot]
            ).start()

        @pl.when(is_act)
        def _():
            pltpu.make_async_copy(
                act_hbm.at[b, pl.ds(base, _CH)], bufs[slot], in_sem.at[slot]
            ).start()

    def wait_in(k, slot):
        b = k // ncr
        base = (k % ncr) * _CH
        pltpu.make_async_copy(
            act_hbm.at[b, pl.ds(base, _CH)], bufs[slot], in_sem.at[slot]
        ).wait()

    def start_out(k, slot):
        b = k // ncr
        base = (k % ncr) * _CH
        pltpu.make_async_copy(
            bufs[slot], out_hbm.at[b, j, pl.ds(base, _CH)], out_sem.at[slot]
        ).start()

    def wait_out(k, slot):
        b = k // ncr
        base = (k % ncr) * _CH
        pltpu.make_async_copy(
            bufs[slot], out_hbm.at[b, j, pl.ds(base, _CH)], out_sem.at[slot]
        ).wait()

    for k in range(_A):
        start_in(k, k % _NBUF)
    for k in range(K):
        slot = k % _NBUF
        wait_in(k, slot)
        start_out(k, slot)
        nxt = k + _A
        if nxt < K:
            prev_out = nxt - _NBUF
            if prev_out >= 0:
                wait_out(prev_out, prev_out % _NBUF)
            start_in(nxt, nxt % _NBUF)
    for k in range(max(0, K - _NBUF), K):
        wait_out(k, k % _NBUF)


def kernel(history, activations, current_idx):
    B, N, H = history.shape
    idx = jnp.asarray(current_idx, dtype=jnp.int32)
    new_idx = idx + 1
    s = new_idx % H
    wrapped = new_idx >= H
    shift = jnp.where(wrapped, s, 0).astype(jnp.int32)
    pos = jnp.where(wrapped, H - 1, idx % H).astype(jnp.int32)
    scalars = jnp.stack([shift, pos])
    scalars16 = jnp.pad(scalars, (0, 14))

    hist_t = jnp.transpose(history, (0, 2, 1))

    mesh = plsc.VectorSubcoreMesh(
        core_axis_name="c", subcore_axis_name="s", num_cores=_NC, num_subcores=_NS
    )
    run = functools.partial(
        pl.kernel,
        out_type=jax.ShapeDtypeStruct((B, H, N), history.dtype),
        mesh=mesh,
        scratch_types=[pltpu.VMEM((16,), jnp.int32)]
        + [pltpu.VMEM((_CH,), history.dtype) for _ in range(_NBUF)]
        + [
            pltpu.SemaphoreType.DMA((_NBUF,)),
            pltpu.SemaphoreType.DMA((_NBUF,)),
        ],
    )(_sc_body)
    out_t = run(scalars16, hist_t, activations)
    return jnp.transpose(out_t, (0, 2, 1))


# speedup vs baseline: 1.3272x; 1.3272x over previous
import functools

import jax
import jax.numpy as jnp
from jax import lax
from jax.experimental import pallas as pl
from jax.experimental.pallas import tpu as pltpu
from jax.experimental.pallas import tpu_sc as plsc

_H = 32
_D = 4
_NC = 2
_NS = 16
_CH = 16384
_NBUF = 6
_A = 3


def _tc_body(scalar_ref, hist_ref, act_ref, out_ref, buf_in, buf_out, in_sem, out_sem):
    shift = scalar_ref[0]
    pos = scalar_ref[1]

    def start_in(j, b):
        src = jax.lax.rem(j + shift, _H)

        @pl.when(j != pos)
        def _():
            pltpu.make_async_copy(
                hist_ref.at[:, src, :], buf_in.at[b], in_sem.at[b]
            ).start()

        @pl.when(j == pos)
        def _():
            pltpu.make_async_copy(act_ref, buf_in.at[b], in_sem.at[b]).start()

    def wait_in(b):
        pltpu.make_async_copy(act_ref, buf_in.at[b], in_sem.at[b]).wait()

    def start_out(j, b):
        pltpu.make_async_copy(
            buf_out.at[b], out_ref.at[:, j, :], out_sem.at[b]
        ).start()

    def wait_out(j, b):
        pltpu.make_async_copy(
            buf_out.at[b], out_ref.at[:, j, :], out_sem.at[b]
        ).wait()

    for j in range(_D):
        start_in(j, j)
    for j in range(_H):
        b = j % _D
        wait_in(b)
        if j >= _D:
            wait_out(j - _D, b)
        buf_out[b] = buf_in[b]
        start_out(j, b)
        if j + _D < _H:
            start_in(j + _D, b)
    for j in range(_H - _D, _H):
        wait_out(j, j % _D)


def _sc_body(scal_hbm, hist_hbm, act_hbm, out_hbm, scal_v, *rest):
    bufs = list(rest[:_NBUF])
    in_sem, out_sem = rest[_NBUF], rest[_NBUF + 1]
    B, H, N = 16, _H, 65536
    ncr = N // _CH
    K = B * ncr

    wid = lax.axis_index("s") * _NC + lax.axis_index("c")
    pltpu.sync_copy(scal_hbm, scal_v)
    sv = scal_v[...]
    shift = sv[0]
    pos = sv[1]

    j = wid
    src = lax.rem(j + shift, _H)
    is_act = j == pos

    def start_in(k, slot):
        b = k // ncr
        base = (k % ncr) * _CH

        @pl.when(jnp.logical_not(is_act))
        def _():
            pltpu.make_async_copy(
                hist_hbm.at[b, src, pl.ds(base, _CH)], bufs[slot], in_sem.at[slot]
            ).start()

        @pl.when(is_act)
        def _():
            pltpu.make_async_copy(
                act_hbm.at[b, pl.ds(base, _CH)], bufs[slot], in_sem.at[slot]
            ).start()

    def wait_in(k, slot):
        b = k // ncr
        base = (k % ncr) * _CH
        pltpu.make_async_copy(
            act_hbm.at[b, pl.ds(base, _CH)], bufs[slot], in_sem.at[slot]
        ).wait()

    def start_out(k, slot):
        b = k // ncr
        base = (k % ncr) * _CH
        pltpu.make_async_copy(
            bufs[slot], out_hbm.at[b, j, pl.ds(base, _CH)], out_sem.at[slot]
        ).start()

    def wait_out(k, slot):
        b = k // ncr
        base = (k % ncr) * _CH
        pltpu.make_async_copy(
            bufs[slot], out_hbm.at[b, j, pl.ds(base, _CH)], out_sem.at[slot]
        ).wait()

    for k in range(_A):
        start_in(k, k % _NBUF)
    for k in range(K):
        slot = k % _NBUF
        wait_in(k, slot)
        start_out(k, slot)
        nxt = k + _A
        if nxt < K:
            prev_out = nxt - _NBUF
            if prev_out >= 0:
                wait_out(prev_out, prev_out % _NBUF)
            start_in(nxt, nxt % _NBUF)
    for k in range(max(0, K - _NBUF), K):
        wait_out(k, k % _NBUF)


def kernel(history, activations, current_idx):
    B, N, H = history.shape
    idx = jnp.asarray(current_idx, dtype=jnp.int32)
    new_idx = idx + 1
    s = new_idx % H
    wrapped = new_idx >= H
    shift = jnp.where(wrapped, s, 0).astype(jnp.int32)
    pos = jnp.where(wrapped, H - 1, idx % H).astype(jnp.int32)
    scalars = jnp.stack([shift, pos])
    scalars16 = jnp.pad(scalars, (0, 14))

    hist_t = jnp.transpose(history, (0, 2, 1))

    out_tc = pl.pallas_call(
        _tc_body,
        grid_spec=pltpu.PrefetchScalarGridSpec(
            num_scalar_prefetch=1,
            grid=(),
            in_specs=[
                pl.BlockSpec(memory_space=pltpu.MemorySpace.HBM),
                pl.BlockSpec(memory_space=pltpu.MemorySpace.HBM),
            ],
            out_specs=pl.BlockSpec(memory_space=pltpu.MemorySpace.HBM),
            scratch_shapes=[
                pltpu.VMEM((_D, B, N), history.dtype),
                pltpu.VMEM((_D, B, N), history.dtype),
                pltpu.SemaphoreType.DMA((_D,)),
                pltpu.SemaphoreType.DMA((_D,)),
            ],
        ),
        out_shape=jax.ShapeDtypeStruct((B, H, N), history.dtype),
    )(scalars, hist_t, activations)

    mesh = plsc.VectorSubcoreMesh(
        core_axis_name="c", subcore_axis_name="s", num_cores=_NC, num_subcores=_NS
    )
    run = functools.partial(
        pl.kernel,
        out_type=jax.ShapeDtypeStruct((B, H, N), history.dtype),
        mesh=mesh,
        scratch_types=[pltpu.VMEM((16,), jnp.int32)]
        + [pltpu.VMEM((_CH,), history.dtype) for _ in range(_NBUF)]
        + [
            pltpu.SemaphoreType.DMA((_NBUF,)),
            pltpu.SemaphoreType.DMA((_NBUF,)),
        ],
    )(_sc_body)
    out_sc = run(scalars16, hist_t, activations)

    out_tc_b, _ = jax.lax.optimization_barrier((out_tc, out_sc))
    return jnp.transpose(out_tc_b, (0, 2, 1))
